# baseline (speedup 1.0000x reference)
import functools

import jax
import jax.numpy as jnp
from jax import lax
from jax.experimental import pallas as pl
from jax.experimental.pallas import tpu as pltpu
from jax.experimental.pallas import tpu_sc as plsc

F32 = jnp.float32


def _cdiv(a, b):
    return -(-a // b)


@functools.lru_cache(maxsize=None)
def _deg_kernel(NROWS, NJ, SPR):
    mesh = plsc.VectorSubcoreMesh(core_axis_name="c", subcore_axis_name="s")
    ZB = SPR // 16
    OB = NROWS // 16

    @functools.partial(
        pl.kernel, mesh=mesh,
        out_type=jax.ShapeDtypeStruct((2, NROWS, 1, 128), F32),
        scratch_types=[
            pltpu.VMEM((1, 128), jnp.int32),
            pltpu.VMEM((128, 1, 128), F32),
            pltpu.VMEM_SHARED((SPR, 1, 128), F32),
        ],
    )
    def deg(ones_hbm, didx_hbm, zeros_hbm, out, dbuf, ones_v, acc):
        c = lax.axis_index("c")
        s = lax.axis_index("s")
        pltpu.sync_copy(zeros_hbm.at[pl.ds(s * ZB, ZB)],
                        acc.at[pl.ds(s * ZB, ZB)])
        pltpu.sync_copy(ones_hbm, ones_v)
        plsc.subcore_barrier()

        def body(j, carry):
            pltpu.sync_copy(didx_hbm.at[c, s, pl.ds(j, 1)], dbuf)
            pltpu.sync_copy(ones_v, acc.at[dbuf.at[0]], add=True)
            return carry

        lax.fori_loop(0, NJ, body, 0, unroll=False)
        plsc.subcore_barrier()
        pltpu.sync_copy(acc.at[pl.ds(s * OB, OB)],
                        out.at[c, pl.ds(s * OB, OB)])

    return deg


@functools.lru_cache(maxsize=None)
def _agg_kernel(NROWS, TR, NJ, SPR):
    mesh = plsc.VectorSubcoreMesh(core_axis_name="c", subcore_axis_name="s")
    ZB = SPR // 16
    OB = NROWS // 16
    assert NJ % 2 == 0

    @functools.partial(
        pl.kernel, mesh=mesh,
        out_type=jax.ShapeDtypeStruct((2, NROWS, 1, 128), F32),
        scratch_types=[
            pltpu.VMEM((2, 128), jnp.int32),
            pltpu.VMEM((2, 128), jnp.int32),
            pltpu.VMEM((128, 1, 128), F32),
            pltpu.VMEM((128, 1, 128), F32),
            pltpu.VMEM_SHARED((SPR, 1, 128), F32),
            pltpu.SemaphoreType.DMA,
            pltpu.SemaphoreType.DMA,
            pltpu.SemaphoreType.DMA,
            pltpu.SemaphoreType.DMA,
        ],
    )
    def agg(tab, idx_hbm, zeros_hbm, out,
            ibuf0, ibuf1, gbuf0, gbuf1, acc, isem0, isem1, gsem0, gsem1):
        c = lax.axis_index("c")
        s = lax.axis_index("s")
        pltpu.sync_copy(zeros_hbm.at[pl.ds(s * ZB, ZB)],
                        acc.at[pl.ds(s * ZB, ZB)])
        plsc.subcore_barrier()

        def pair(p, carry):
            ld0 = pltpu.async_copy(idx_hbm.at[c, s, 2 * p], ibuf0, isem0)
            ld1 = pltpu.async_copy(idx_hbm.at[c, s, 2 * p + 1], ibuf1, isem1)
            ld0.wait()
            g0 = pltpu.async_copy(tab.at[ibuf0.at[0]], gbuf0, gsem0)
            ld1.wait()
            g1 = pltpu.async_copy(tab.at[ibuf1.at[0]], gbuf1, gsem1)
            g0.wait()
            pltpu.sync_copy(gbuf0, acc.at[ibuf0.at[1]], add=True)
            g1.wait()
            pltpu.sync_copy(gbuf1, acc.at[ibuf1.at[1]], add=True)
            return carry

        lax.fori_loop(0, NJ // 2, pair, 0, unroll=False)
        plsc.subcore_barrier()
        pltpu.sync_copy(acc.at[pl.ds(s * OB, OB)],
                        out.at[c, pl.ds(s * OB, OB)])

    return agg


def _p0_call(D, x, BM=1000):
    N, DF = x.shape
    nb = N // BM

    def body(d_ref, x_ref, dv_ref, xs_ref):
        deg = d_ref[0, :, 0, :] + d_ref[1, :, 0, :]
        dinv = lax.rsqrt(jnp.maximum(deg, 1.0))[:, 0:1]
        dv_ref[...] = jnp.broadcast_to(dinv, (BM, 128))
        xs_ref[...] = (x_ref[...] * dinv)[:, None, :]

    return pl.pallas_call(
        body,
        grid=(nb,),
        in_specs=[
            pl.BlockSpec((2, BM, 1, 128), lambda i: (0, i, 0, 0)),
            pl.BlockSpec((BM, DF), lambda i: (i, 0)),
        ],
        out_specs=[
            pl.BlockSpec((BM, 128), lambda i: (i, 0)),
            pl.BlockSpec((BM, 1, DF), lambda i: (i, 0, 0)),
        ],
        out_shape=[jax.ShapeDtypeStruct((N, 128), F32),
                   jax.ShapeDtypeStruct((N, 1, DF), F32)],
    )(D, x)


def _p1_call(S, dv, W, b, sum_parts, BM=1000):
    N = S.shape[1]
    K, H = W.shape
    nb = N // BM

    def body(s_ref, dv_ref, w_ref, b_ref, u_ref, st_ref, acc):
        i = pl.program_id(0)
        dinv = dv_ref[:, 0:1]
        if sum_parts:
            aggv = (s_ref[0, :, 0, :] + s_ref[1, :, 0, :]) * dinv
            u = jnp.dot(aggv, w_ref[...], preferred_element_type=F32)
        else:
            u = jnp.dot(s_ref[0, :, 0, :] * dinv, w_ref[0:128, :],
                        preferred_element_type=F32)
            u += jnp.dot(s_ref[1, :, 0, :] * dinv, w_ref[128:256, :],
                         preferred_element_type=F32)
        u = u + b_ref[...]
        u_ref[...] = u

        @pl.when(i == 0)
        def _():
            acc[...] = jnp.zeros((8, H), F32)

        acc[0:1, :] += jnp.sum(u, axis=0, keepdims=True)
        acc[1:2, :] += jnp.sum(u * u, axis=0, keepdims=True)

        @pl.when(i == nb - 1)
        def _():
            st_ref[...] = acc[...]

    return pl.pallas_call(
        body,
        grid=(nb,),
        in_specs=[
            pl.BlockSpec((2, BM, 1, 128), lambda i: (0, i, 0, 0)),
            pl.BlockSpec((BM, 128), lambda i: (i, 0)),
            pl.BlockSpec((K, H), lambda i: (0, 0)),
            pl.BlockSpec((1, H), lambda i: (0, 0)),
        ],
        out_specs=[
            pl.BlockSpec((BM, H), lambda i: (i, 0)),
            pl.BlockSpec((8, H), lambda i: (0, 0)),
        ],
        out_shape=[jax.ShapeDtypeStruct((N, H), F32),
                   jax.ShapeDtypeStruct((8, H), F32)],
        scratch_shapes=[pltpu.VMEM((8, H), F32)],
    )(S, dv, W, b)


def _p2_call(u, st, g, be, dv, scale, BM=1000):
    N, H = u.shape
    nb = N // BM

    def body(u_ref, st_ref, g_ref, be_ref, dv_ref, q_ref):
        stats = st_ref[...]
        m = stats[0:1, :] * (1.0 / N)
        var = stats[1:2, :] * (1.0 / N) - m * m
        a = g_ref[...] * lax.rsqrt(var + 1e-5)
        cc = be_ref[...] - m * a
        h = jnp.maximum(u_ref[...] * a + cc, 0.0)
        if scale:
            h = h * dv_ref[:, 0:1]
        q_ref[...] = h[:, None, :]

    return pl.pallas_call(
        body,
        grid=(nb, 2),
        in_specs=[
            pl.BlockSpec((BM, 128), lambda i, h: (i, h)),
            pl.BlockSpec((8, 128), lambda i, h: (0, h)),
            pl.BlockSpec((1, 128), lambda i, h: (0, h)),
            pl.BlockSpec((1, 128), lambda i, h: (0, h)),
            pl.BlockSpec((BM, 128), lambda i, h: (i, 0)),
        ],
        out_specs=pl.BlockSpec((BM, 1, 128), lambda i, h: (h * nb + i, 0, 0)),
        out_shape=jax.ShapeDtypeStruct((2 * N, 1, 128), F32),
    )(u, st, g, be, dv)


def _pool_call(q2, batch2, Wc1, bc1, Wc2, bc2, G, BM=1000):
    N2 = q2.shape[0]
    N = N2 // 2
    H = 256
    HID = Wc1.shape[1]
    T = Wc2.shape[1]
    nb = N // BM

    def body(qa_ref, qb_ref, b_ref, w1_ref, b1_ref, w2_ref, b2_ref,
             out_ref, acc, cnt):
        i = pl.program_id(0)

        @pl.when(i == 0)
        def _():
            acc[...] = jnp.zeros((128, H), F32)
            cnt[...] = jnp.zeros((128, 8), F32)

        h = jnp.concatenate([qa_ref[:, 0, :], qb_ref[:, 0, :]], axis=1)
        gids = lax.broadcasted_iota(jnp.int32, (BM, 128), 1)
        onehot = (b_ref[...] == gids).astype(F32)
        acc[...] += lax.dot_general(onehot, h, (((0,), (0,)), ((), ())),
                                    preferred_element_type=F32)
        cnt[...] += lax.dot_general(onehot, jnp.ones((BM, 8), F32),
                                    (((0,), (0,)), ((), ())),
                                    preferred_element_type=F32)

        @pl.when(i == nb - 1)
        def _():
            inv = 1.0 / jnp.maximum(cnt[:, 0:1], 1.0)
            emb = acc[...] * inv
            hid = jnp.maximum(
                jnp.dot(emb, w1_ref[...], preferred_element_type=F32)
                + b1_ref[...], 0.0)
            logits = (jnp.dot(hid, w2_ref[...], preferred_element_type=F32)
                      + b2_ref[...])
            out_ref[...] = logits[0:G, :]

    return pl.pallas_call(
        body,
        grid=(nb,),
        in_specs=[
            pl.BlockSpec((BM, 1, 128), lambda i: (i, 0, 0)),
            pl.BlockSpec((BM, 1, 128), lambda i: (nb + i, 0, 0)),
            pl.BlockSpec((BM, 1), lambda i: (i, 0)),
            pl.BlockSpec((H, HID), lambda i: (0, 0)),
            pl.BlockSpec((1, HID), lambda i: (0, 0)),
            pl.BlockSpec((HID, T), lambda i: (0, 0)),
            pl.BlockSpec((1, T), lambda i: (0, 0)),
        ],
        out_specs=pl.BlockSpec((G, T), lambda i: (0, 0)),
        out_shape=jax.ShapeDtypeStruct((G, T), F32),
        scratch_shapes=[pltpu.VMEM((128, H), F32),
                        pltpu.VMEM((128, 8), F32)],
    )(q2, q2, batch2, Wc1, bc1, Wc2, bc2)


def _pack_idx(idx, fill, E):
    ec = E // 2
    tot = ec // 128
    nj = _cdiv(tot, 16)
    ch = idx.reshape(2, tot, 128)
    pad = jnp.full((2, nj * 16 - tot, 128), fill, jnp.int32)
    return jnp.concatenate([ch, pad], axis=1).reshape(2, 16, nj, 128), nj


def _pack_pair(src, dst, split, E, N):
    if split:
        tot = (E // 2) // 128
        nj = _cdiv(tot, 16)
        nj += (-nj) % 4
        s_ch = src.reshape(2, tot, 128)
        d_ch = dst.reshape(2, tot, 128)
        s_ch = jnp.concatenate(
            [s_ch, jnp.zeros((2, nj * 16 - tot, 128), jnp.int32)], axis=1)
        d_ch = jnp.concatenate(
            [d_ch, jnp.full((2, nj * 16 - tot, 128), N, jnp.int32)], axis=1)
        both = jnp.stack([s_ch, d_ch], axis=2)
        return both.reshape(2, 16, nj, 2, 128), nj
    tot = E // 128
    nj = _cdiv(tot, 16)
    nj += (-nj) % 4
    s_ch = jnp.concatenate(
        [src.reshape(tot, 128),
         jnp.zeros((nj * 16 - tot, 128), jnp.int32)], axis=0)
    d_ch = jnp.concatenate(
        [dst.reshape(tot, 128),
         jnp.full((nj * 16 - tot, 128), N, jnp.int32)], axis=0)
    both = jnp.stack([s_ch, d_ch], axis=1).reshape(16, nj, 2, 128)
    return jnp.stack([both, both + jnp.array([N, 0], jnp.int32)[:, None]]), nj


def _impl(x, edge_index, batch, W_in, b_in, g_in, be_in, Wm, bm, gm, bem,
          Wc1, bc1, Wc2, bc2):
    N, DF = x.shape
    E = edge_index.shape[1]
    H = W_in.shape[1]
    L = Wm.shape[0]
    G = 64
    src = edge_index[0]
    dst = edge_index[1]
    SPR = _cdiv(N + 1, 16) * 16

    idxB, njB = _pack_pair(src, dst, True, E, N)
    idxA, njA = _pack_pair(src, dst, False, E, N)
    didxD, njD = _pack_idx(dst, N, E)

    zeros128 = jnp.zeros((SPR, 1, 128), F32)
    ones128 = jnp.ones((128, 1, 128), F32)

    D = _deg_kernel(N, njD, SPR)(ones128, didxD, zeros128)
    dv, xs = _p0_call(D, x)

    aggB = _agg_kernel(N, N, njB, SPR)
    aggA = _agg_kernel(N, 2 * N, njA, SPR)

    S = aggB(xs, idxB, zeros128)
    u, st = _p1_call(S, dv, W_in, b_in.reshape(1, H), sum_parts=True)
    q2 = _p2_call(u, st, g_in.reshape(1, H), be_in.reshape(1, H), dv,
                  scale=True)

    for i in range(L):
        S = aggA(q2, idxA, zeros128)
        u, st = _p1_call(S, dv, Wm[i], bm[i].reshape(1, H), sum_parts=False)
        q2 = _p2_call(u, st, gm[i].reshape(1, H), bem[i].reshape(1, H),
                      dv, scale=(i < L - 1))

    return _pool_call(q2, batch.reshape(N, 1).astype(jnp.int32),
                      Wc1, bc1.reshape(1, Wc1.shape[1]),
                      Wc2, bc2.reshape(1, Wc2.shape[1]), G)


kernel = jax.jit(_impl)


# speedup vs baseline: 1.3281x; 1.3281x over previous
import functools

import jax
import jax.numpy as jnp
from jax import lax
from jax.experimental import pallas as pl
from jax.experimental.pallas import tpu as pltpu
from jax.experimental.pallas import tpu_sc as plsc

F32 = jnp.float32


def _cdiv(a, b):
    return -(-a // b)


@functools.lru_cache(maxsize=None)
def _deg_kernel(NROWS, NJ, SPR):
    mesh = plsc.VectorSubcoreMesh(core_axis_name="c", subcore_axis_name="s")
    ZB = SPR // 16
    OB = NROWS // 16

    @functools.partial(
        pl.kernel, mesh=mesh,
        out_type=jax.ShapeDtypeStruct((2, NROWS, 1, 128), F32),
        scratch_types=[
            pltpu.VMEM((1, 128), jnp.int32),
            pltpu.VMEM((128, 1, 128), F32),
            pltpu.VMEM_SHARED((SPR, 1, 128), F32),
        ],
    )
    def deg(ones_hbm, didx_hbm, zeros_hbm, out, dbuf, ones_v, acc):
        c = lax.axis_index("c")
        s = lax.axis_index("s")
        pltpu.sync_copy(zeros_hbm.at[pl.ds(s * ZB, ZB)],
                        acc.at[pl.ds(s * ZB, ZB)])
        pltpu.sync_copy(ones_hbm, ones_v)
        plsc.subcore_barrier()

        def body(j, carry):
            pltpu.sync_copy(didx_hbm.at[c, s, pl.ds(j, 1)], dbuf)
            pltpu.sync_copy(ones_v, acc.at[dbuf.at[0]], add=True)
            return carry

        lax.fori_loop(0, NJ, body, 0, unroll=False)
        plsc.subcore_barrier()
        pltpu.sync_copy(acc.at[pl.ds(s * OB, OB)],
                        out.at[c, pl.ds(s * OB, OB)])

    return deg


@functools.lru_cache(maxsize=None)
def _agg_kernel(NROWS, TR, NJ, SPR):
    mesh = plsc.VectorSubcoreMesh(core_axis_name="c", subcore_axis_name="s")
    ZB = SPR // 16
    OB = NROWS // 16
    assert NJ % 2 == 0

    @functools.partial(
        pl.kernel, mesh=mesh,
        out_type=jax.ShapeDtypeStruct((2, NROWS, 1, 128), F32),
        scratch_types=[
            pltpu.VMEM((2, 128), jnp.int32),
            pltpu.VMEM((2, 128), jnp.int32),
            pltpu.VMEM((128, 1, 128), F32),
            pltpu.VMEM((128, 1, 128), F32),
            pltpu.VMEM_SHARED((SPR, 1, 128), F32),
            pltpu.SemaphoreType.DMA,
            pltpu.SemaphoreType.DMA,
            pltpu.SemaphoreType.DMA,
            pltpu.SemaphoreType.DMA,
        ],
    )
    def agg(tab, idx_hbm, zeros_hbm, out,
            ibuf0, ibuf1, gbuf0, gbuf1, acc, isem0, isem1, gsem0, gsem1):
        c = lax.axis_index("c")
        s = lax.axis_index("s")
        pltpu.sync_copy(zeros_hbm.at[pl.ds(s * ZB, ZB)],
                        acc.at[pl.ds(s * ZB, ZB)])
        plsc.subcore_barrier()

        def pair(p, carry):
            ld0 = pltpu.async_copy(idx_hbm.at[c, s, 2 * p], ibuf0, isem0)
            ld1 = pltpu.async_copy(idx_hbm.at[c, s, 2 * p + 1], ibuf1, isem1)
            ld0.wait()
            g0 = pltpu.async_copy(tab.at[ibuf0.at[0]], gbuf0, gsem0)
            ld1.wait()
            g1 = pltpu.async_copy(tab.at[ibuf1.at[0]], gbuf1, gsem1)
            g0.wait()
            pltpu.sync_copy(gbuf0, acc.at[ibuf0.at[1]], add=True)
            g1.wait()
            pltpu.sync_copy(gbuf1, acc.at[ibuf1.at[1]], add=True)
            return carry

        lax.fori_loop(0, NJ // 2, pair, 0, unroll=False)
        plsc.subcore_barrier()
        pltpu.sync_copy(acc.at[pl.ds(s * OB, OB)],
                        out.at[c, pl.ds(s * OB, OB)])

    return agg


def _p0_call(D, x, BM=1000):
    N, DF = x.shape
    nb = N // BM

    def body(d_ref, x_ref, dv_ref, xs_ref):
        deg = d_ref[0, :, 0, :] + d_ref[1, :, 0, :]
        dinv = lax.rsqrt(jnp.maximum(deg, 1.0))[:, 0:1]
        dv_ref[...] = jnp.broadcast_to(dinv, (BM, 128))
        xs_ref[...] = (x_ref[...] * dinv)[:, None, :]

    return pl.pallas_call(
        body,
        grid=(nb,),
        in_specs=[
            pl.BlockSpec((2, BM, 1, 128), lambda i: (0, i, 0, 0)),
            pl.BlockSpec((BM, DF), lambda i: (i, 0)),
        ],
        out_specs=[
            pl.BlockSpec((BM, 128), lambda i: (i, 0)),
            pl.BlockSpec((BM, 1, DF), lambda i: (i, 0, 0)),
        ],
        out_shape=[jax.ShapeDtypeStruct((N, 128), F32),
                   jax.ShapeDtypeStruct((N, 1, DF), F32)],
    )(D, x)


def _p1_call(S, dv, W, b, sum_parts, BM=1000):
    N = S.shape[1]
    K, H = W.shape
    nb = N // BM

    def body(s_ref, dv_ref, w_ref, b_ref, u_ref, st_ref, acc):
        i = pl.program_id(0)
        dinv = dv_ref[:, 0:1]
        if sum_parts:
            aggv = (s_ref[0, :, 0, :] + s_ref[1, :, 0, :]) * dinv
            u = jnp.dot(aggv, w_ref[...], preferred_element_type=F32)
        else:
            u = jnp.dot(s_ref[0, :, 0, :] * dinv, w_ref[0:128, :],
                        preferred_element_type=F32)
            u += jnp.dot(s_ref[1, :, 0, :] * dinv, w_ref[128:256, :],
                         preferred_element_type=F32)
        u = u + b_ref[...]
        u_ref[...] = u

        @pl.when(i == 0)
        def _():
            acc[...] = jnp.zeros((8, H), F32)

        acc[0:1, :] += jnp.sum(u, axis=0, keepdims=True)
        acc[1:2, :] += jnp.sum(u * u, axis=0, keepdims=True)

        @pl.when(i == nb - 1)
        def _():
            st_ref[...] = acc[...]

    return pl.pallas_call(
        body,
        grid=(nb,),
        in_specs=[
            pl.BlockSpec((2, BM, 1, 128), lambda i: (0, i, 0, 0)),
            pl.BlockSpec((BM, 128), lambda i: (i, 0)),
            pl.BlockSpec((K, H), lambda i: (0, 0)),
            pl.BlockSpec((1, H), lambda i: (0, 0)),
        ],
        out_specs=[
            pl.BlockSpec((BM, H), lambda i: (i, 0)),
            pl.BlockSpec((8, H), lambda i: (0, 0)),
        ],
        out_shape=[jax.ShapeDtypeStruct((N, H), F32),
                   jax.ShapeDtypeStruct((8, H), F32)],
        scratch_shapes=[pltpu.VMEM((8, H), F32)],
    )(S, dv, W, b)


def _p2_call(u, st, g, be, dv, scale, BM=1000):
    N, H = u.shape
    nb = N // BM

    def body(u_ref, st_ref, g_ref, be_ref, dv_ref, q_ref):
        stats = st_ref[...]
        m = stats[0:1, :] * (1.0 / N)
        var = stats[1:2, :] * (1.0 / N) - m * m
        a = g_ref[...] * lax.rsqrt(var + 1e-5)
        cc = be_ref[...] - m * a
        h = jnp.maximum(u_ref[...] * a + cc, 0.0)
        if scale:
            h = h * dv_ref[:, 0:1]
        q_ref[...] = h[:, None, :]

    return pl.pallas_call(
        body,
        grid=(nb, 2),
        in_specs=[
            pl.BlockSpec((BM, 128), lambda i, h: (i, h)),
            pl.BlockSpec((8, 128), lambda i, h: (0, h)),
            pl.BlockSpec((1, 128), lambda i, h: (0, h)),
            pl.BlockSpec((1, 128), lambda i, h: (0, h)),
            pl.BlockSpec((BM, 128), lambda i, h: (i, 0)),
        ],
        out_specs=pl.BlockSpec((BM, 1, 128), lambda i, h: (h * nb + i, 0, 0)),
        out_shape=jax.ShapeDtypeStruct((2 * N, 1, 128), F32),
    )(u, st, g, be, dv)


def _pool_call(q2, batch2, Wc1, bc1, Wc2, bc2, G, BM=1000):
    N2 = q2.shape[0]
    N = N2 // 2
    H = 256
    HID = Wc1.shape[1]
    T = Wc2.shape[1]
    nb = N // BM

    def body(qa_ref, qb_ref, b_ref, w1_ref, b1_ref, w2_ref, b2_ref,
             out_ref, acc, cnt):
        i = pl.program_id(0)

        @pl.when(i == 0)
        def _():
            acc[...] = jnp.zeros((128, H), F32)
            cnt[...] = jnp.zeros((128, 8), F32)

        h = jnp.concatenate([qa_ref[:, 0, :], qb_ref[:, 0, :]], axis=1)
        gids = lax.broadcasted_iota(jnp.int32, (BM, 128), 1)
        onehot = (b_ref[...] == gids).astype(F32)
        acc[...] += lax.dot_general(onehot, h, (((0,), (0,)), ((), ())),
                                    preferred_element_type=F32)
        cnt[...] += lax.dot_general(onehot, jnp.ones((BM, 8), F32),
                                    (((0,), (0,)), ((), ())),
                                    preferred_element_type=F32)

        @pl.when(i == nb - 1)
        def _():
            inv = 1.0 / jnp.maximum(cnt[:, 0:1], 1.0)
            emb = acc[...] * inv
            hid = jnp.maximum(
                jnp.dot(emb, w1_ref[...], preferred_element_type=F32)
                + b1_ref[...], 0.0)
            logits = (jnp.dot(hid, w2_ref[...], preferred_element_type=F32)
                      + b2_ref[...])
            out_ref[...] = logits[0:G, :]

    return pl.pallas_call(
        body,
        grid=(nb,),
        in_specs=[
            pl.BlockSpec((BM, 1, 128), lambda i: (i, 0, 0)),
            pl.BlockSpec((BM, 1, 128), lambda i: (nb + i, 0, 0)),
            pl.BlockSpec((BM, 1), lambda i: (i, 0)),
            pl.BlockSpec((H, HID), lambda i: (0, 0)),
            pl.BlockSpec((1, HID), lambda i: (0, 0)),
            pl.BlockSpec((HID, T), lambda i: (0, 0)),
            pl.BlockSpec((1, T), lambda i: (0, 0)),
        ],
        out_specs=pl.BlockSpec((G, T), lambda i: (0, 0)),
        out_shape=jax.ShapeDtypeStruct((G, T), F32),
        scratch_shapes=[pltpu.VMEM((128, H), F32),
                        pltpu.VMEM((128, 8), F32)],
    )(q2, q2, batch2, Wc1, bc1, Wc2, bc2)


def _pack_idx(idx, fill, E):
    ec = E // 2
    tot = ec // 128
    nj = _cdiv(tot, 16)
    ch = idx.reshape(2, tot, 128)
    padd = fill + (jnp.arange(128, dtype=jnp.int32) % 16)
    pad = jnp.broadcast_to(padd, (2, nj * 16 - tot, 128))
    return jnp.concatenate([ch, pad], axis=1).reshape(2, 16, nj, 128), nj


def _pack_pair(src, dst, split, E, N):
    if split:
        tot = (E // 2) // 128
        nj = _cdiv(tot, 16)
        nj += nj % 2
        s_ch = src.reshape(2, tot, 128)
        d_ch = dst.reshape(2, tot, 128)
        padd = N + (jnp.arange(128, dtype=jnp.int32) % 16)
        s_ch = jnp.concatenate(
            [s_ch, jnp.zeros((2, nj * 16 - tot, 128), jnp.int32)], axis=1)
        d_ch = jnp.concatenate(
            [d_ch, jnp.broadcast_to(padd, (2, nj * 16 - tot, 128))], axis=1)
        both = jnp.stack([s_ch, d_ch], axis=2)
        return both.reshape(2, 16, nj, 2, 128), nj
    tot = E // 128
    nj = _cdiv(tot, 16)
    nj += nj % 2
    padd = N + (jnp.arange(128, dtype=jnp.int32) % 16)
    s_ch = jnp.concatenate(
        [src.reshape(tot, 128),
         jnp.zeros((nj * 16 - tot, 128), jnp.int32)], axis=0)
    d_ch = jnp.concatenate(
        [dst.reshape(tot, 128),
         jnp.broadcast_to(padd, (nj * 16 - tot, 128))], axis=0)
    both = jnp.stack([s_ch, d_ch], axis=1).reshape(16, nj, 2, 128)
    return jnp.stack([both, both + jnp.array([N, 0], jnp.int32)[:, None]]), nj


def _impl(x, edge_index, batch, W_in, b_in, g_in, be_in, Wm, bm, gm, bem,
          Wc1, bc1, Wc2, bc2):
    N, DF = x.shape
    E = edge_index.shape[1]
    H = W_in.shape[1]
    L = Wm.shape[0]
    G = 64
    src = edge_index[0]
    dst = edge_index[1]
    SPR = _cdiv(N + 1, 16) * 16

    idxB, njB = _pack_pair(src, dst, True, E, N)
    idxA, njA = _pack_pair(src, dst, False, E, N)
    didxD, njD = _pack_idx(dst, N, E)

    zeros128 = jnp.zeros((SPR, 1, 128), F32)
    ones128 = jnp.ones((128, 1, 128), F32)

    D = _deg_kernel(N, njD, SPR)(ones128, didxD, zeros128)
    dv, xs = _p0_call(D, x)

    aggB = _agg_kernel(N, N, njB, SPR)
    aggA = _agg_kernel(N, 2 * N, njA, SPR)

    S = aggB(xs, idxB, zeros128)
    u, st = _p1_call(S, dv, W_in, b_in.reshape(1, H), sum_parts=True)
    q2 = _p2_call(u, st, g_in.reshape(1, H), be_in.reshape(1, H), dv,
                  scale=True)

    for i in range(L):
        S = aggA(q2, idxA, zeros128)
        u, st = _p1_call(S, dv, Wm[i], bm[i].reshape(1, H), sum_parts=False)
        q2 = _p2_call(u, st, gm[i].reshape(1, H), bem[i].reshape(1, H),
                      dv, scale=(i < L - 1))

    return _pool_call(q2, batch.reshape(N, 1).astype(jnp.int32),
                      Wc1, bc1.reshape(1, Wc1.shape[1]),
                      Wc2, bc2.reshape(1, Wc2.shape[1]), G)


kernel = jax.jit(_impl)
